# baseline (speedup 1.0000x reference)
import functools

import jax
import jax.numpy as jnp
from jax import lax
from jax.experimental import pallas as pl
from jax.experimental.pallas import tpu as pltpu
from jax.experimental.pallas import tpu_sc as plsc

B = 16384
MF = 32
MLPH = 16
MLP = 2 * MLPH
NROWS = 1000000
NTILES = NROWS // 8
NC, NS = 2, 16
NW = NC * NS
BPW = B // NW
NB = 16
NG = BPW // NB


def _sc_gather_body(user_hbm, item_hbm, mfu_hbm, mfi_hbm, mlu_hbm, mli_hbm,
                    xmf_hbm, xmlpu_hbm, xmlpi_hbm,
                    uidx, iidx, t_mfu, t_mfi, t_mlu, t_mli,
                    r_mf, r_mlu, r_mli, sem):
    wid = lax.axis_index("s") * NC + lax.axis_index("c")
    base = wid * BPW
    pltpu.sync_copy(user_hbm.at[pl.ds(base, BPW)], uidx)
    pltpu.sync_copy(item_hbm.at[pl.ds(base, BPW)], iidx)
    lanes = lax.iota(jnp.int32, 16)

    def group(g, carry):
        vu = uidx[pl.ds(g * NB, NB)]
        vi = iidx[pl.ds(g * NB, NB)]
        copies = []
        subrow = []
        for l in range(NB):
            su = jnp.sum(jnp.where(lanes == l, vu, 0))
            si = jnp.sum(jnp.where(lanes == l, vi, 0))
            tu, ru = su // 8, su % 8
            ti, ri = si // 8, si % 8
            subrow.append((ru, ri))
            ou = pl.multiple_of(tu * 8, 8)
            oi = pl.multiple_of(ti * 8, 8)
            du = pl.ds(ou, 8)
            di = pl.ds(oi, 8)
            copies.append(pltpu.async_copy(mfu_hbm.at[du], t_mfu.at[l], sem))
            copies.append(pltpu.async_copy(mfi_hbm.at[di], t_mfi.at[l], sem))
            copies.append(pltpu.async_copy(mlu_hbm.at[du], t_mlu.at[l], sem))
            copies.append(pltpu.async_copy(mli_hbm.at[di], t_mli.at[l], sem))
        for c in copies:
            c.wait()
        for l in range(NB):
            ru, ri = subrow[l]
            for d in (0, 16):
                r_mf[l, pl.ds(d, 16)] = (t_mfu[l, ru, pl.ds(d, 16)]
                                         * t_mfi[l, ri, pl.ds(d, 16)])
            r_mlu[l, pl.ds(0, 16)] = t_mlu[l, ru, pl.ds(0, 16)]
            r_mli[l, pl.ds(0, 16)] = t_mli[l, ri, pl.ds(0, 16)]
        out = pl.ds(base + g * NB, NB)
        pltpu.sync_copy(r_mf, xmf_hbm.at[out])
        pltpu.sync_copy(r_mlu, xmlpu_hbm.at[out])
        pltpu.sync_copy(r_mli, xmlpi_hbm.at[out])
        return carry

    lax.fori_loop(0, NG, group, 0)


_sc_gather = functools.partial(
    pl.kernel,
    out_type=(jax.ShapeDtypeStruct((B, MF), jnp.float32),
              jax.ShapeDtypeStruct((B, MLPH), jnp.float32),
              jax.ShapeDtypeStruct((B, MLPH), jnp.float32)),
    mesh=plsc.VectorSubcoreMesh(core_axis_name="c", subcore_axis_name="s"),
    scratch_types=[
        pltpu.VMEM((BPW,), jnp.int32),
        pltpu.VMEM((BPW,), jnp.int32),
        pltpu.VMEM((NB, 8, MF), jnp.float32),
        pltpu.VMEM((NB, 8, MF), jnp.float32),
        pltpu.VMEM((NB, 8, MLPH), jnp.float32),
        pltpu.VMEM((NB, 8, MLPH), jnp.float32),
        pltpu.VMEM((NB, MF), jnp.float32),
        pltpu.VMEM((NB, MLPH), jnp.float32),
        pltpu.VMEM((NB, MLPH), jnp.float32),
        pltpu.SemaphoreType.DMA,
    ],
    compiler_params=pltpu.CompilerParams(needs_layout_passes=False),
)(_sc_gather_body)


def _tc_mlp_body(xmf_ref, xmlpu_ref, xmlpi_ref, w_ref, b_ref, wf_ref, bf_ref,
                 out_ref):
    x = jnp.concatenate([xmlpu_ref[...], xmlpi_ref[...]], axis=1)
    for i in range(4):
        x = jnp.maximum(
            jnp.dot(x, w_ref[i], preferred_element_type=jnp.float32) + b_ref[i],
            0.0)
    wf = wf_ref[...]
    s = (jnp.sum(xmf_ref[...] * wf[:, :MF], axis=1, keepdims=True)
         + jnp.sum(x * wf[:, MF:], axis=1, keepdims=True)
         + bf_ref[0, 0])
    out_ref[...] = 1.0 / (1.0 + jnp.exp(-s))


def kernel(user, item, mf_user_embed, mf_item_embed, mlp_user_embed,
           mlp_item_embed, W0, b0, W1, b1, W2, b2, W3, b3, Wf, bf):
    xmf, xmlpu, xmlpi = _sc_gather(
        user.astype(jnp.int32), item.astype(jnp.int32),
        mf_user_embed, mf_item_embed, mlp_user_embed, mlp_item_embed)
    Ws = jnp.stack([W0, W1, W2, W3])
    bs = jnp.stack([b0, b1, b2, b3]).reshape(4, 1, MLP)
    wf = Wf.reshape(1, MF + MLP)
    out = pl.pallas_call(
        _tc_mlp_body,
        out_shape=jax.ShapeDtypeStruct((B, 1), jnp.float32),
    )(xmf, xmlpu, xmlpi, Ws, bs, wf, bf.reshape(1, 1))
    return out


# speedup vs baseline: 1.0008x; 1.0008x over previous
import functools

import jax
import jax.numpy as jnp
from jax import lax
from jax.experimental import pallas as pl
from jax.experimental.pallas import tpu as pltpu
from jax.experimental.pallas import tpu_sc as plsc

B = 16384
MF = 32
MLPH = 16
MLP = 2 * MLPH
NROWS = 1000000
NTILES = NROWS // 8
NC, NS = 2, 16
NW = NC * NS
BPW = B // NW
NB = 16
NG = BPW // NB


def _sc_gather_body(user_hbm, item_hbm, mfu_hbm, mfi_hbm, mlu_hbm, mli_hbm,
                    xmf_hbm, xmlpu_hbm, xmlpi_hbm,
                    uidx, iidx, t_mfu, t_mfi, t_mlu, t_mli,
                    r_mf, r_mlu, r_mli, sem):
    wid = lax.axis_index("s") * NC + lax.axis_index("c")
    base = wid * BPW
    pltpu.sync_copy(user_hbm.at[pl.ds(base, BPW)], uidx)
    pltpu.sync_copy(item_hbm.at[pl.ds(base, BPW)], iidx)
    lanes = lax.iota(jnp.int32, 16)

    def group(g, carry):
        vu = uidx[pl.ds(g * NB, NB)]
        vi = iidx[pl.ds(g * NB, NB)]
        copies = []
        subrow = []
        for l in range(NB):
            su = jnp.sum(jnp.where(lanes == l, vu, 0))
            si = jnp.sum(jnp.where(lanes == l, vi, 0))
            tu, ru = su // 8, su % 8
            ti, ri = si // 8, si % 8
            subrow.append((ru, ri))
            ou = pl.multiple_of(tu * 8, 8)
            oi = pl.multiple_of(ti * 8, 8)
            du = pl.ds(ou, 8)
            di = pl.ds(oi, 8)
            copies.append(pltpu.async_copy(mfu_hbm.at[du], t_mfu.at[l], sem))
            copies.append(pltpu.async_copy(mfi_hbm.at[di], t_mfi.at[l], sem))
            copies.append(pltpu.async_copy(mlu_hbm.at[du], t_mlu.at[l], sem))
            copies.append(pltpu.async_copy(mli_hbm.at[di], t_mli.at[l], sem))
        for c in copies:
            c.wait()
        for l in range(NB):
            ru, ri = subrow[l]
            for d in (0, 16):
                r_mf[l, pl.ds(d, 16)] = (t_mfu[l, ru, pl.ds(d, 16)]
                                         * t_mfi[l, ri, pl.ds(d, 16)])
            r_mlu[l, pl.ds(0, 16)] = t_mlu[l, ru, pl.ds(0, 16)]
            r_mli[l, pl.ds(0, 16)] = t_mli[l, ri, pl.ds(0, 16)]
        out = pl.ds(pl.multiple_of(base + g * NB, NB), NB)
        pltpu.sync_copy(r_mf, xmf_hbm.at[out])
        pltpu.sync_copy(r_mlu, xmlpu_hbm.at[out])
        pltpu.sync_copy(r_mli, xmlpi_hbm.at[out])
        return carry

    lax.fori_loop(0, NG, group, 0)


_sc_gather = functools.partial(
    pl.kernel,
    out_type=(jax.ShapeDtypeStruct((B, MF), jnp.float32),
              jax.ShapeDtypeStruct((B, MLPH), jnp.float32),
              jax.ShapeDtypeStruct((B, MLPH), jnp.float32)),
    mesh=plsc.VectorSubcoreMesh(core_axis_name="c", subcore_axis_name="s"),
    scratch_types=[
        pltpu.VMEM((BPW,), jnp.int32),
        pltpu.VMEM((BPW,), jnp.int32),
        pltpu.VMEM((NB, 8, MF), jnp.float32),
        pltpu.VMEM((NB, 8, MF), jnp.float32),
        pltpu.VMEM((NB, 8, MLPH), jnp.float32),
        pltpu.VMEM((NB, 8, MLPH), jnp.float32),
        pltpu.VMEM((NB, MF), jnp.float32),
        pltpu.VMEM((NB, MLPH), jnp.float32),
        pltpu.VMEM((NB, MLPH), jnp.float32),
        pltpu.SemaphoreType.DMA,
    ],
    compiler_params=pltpu.CompilerParams(needs_layout_passes=False,
                                         use_tc_tiling_on_sc=True),
)(_sc_gather_body)


def _tc_mlp_body(xmf_ref, xmlpu_ref, xmlpi_ref, w_ref, b_ref, wf_ref, bf_ref,
                 out_ref):
    x = jnp.concatenate([xmlpu_ref[...], xmlpi_ref[...]], axis=1)
    for i in range(4):
        x = jnp.maximum(
            jnp.dot(x, w_ref[i], preferred_element_type=jnp.float32) + b_ref[i],
            0.0)
    wf = wf_ref[...]
    s = (jnp.sum(xmf_ref[...] * wf[:, :MF], axis=1, keepdims=True)
         + jnp.sum(x * wf[:, MF:], axis=1, keepdims=True)
         + bf_ref[0, 0])
    out_ref[...] = 1.0 / (1.0 + jnp.exp(-s))


def kernel(user, item, mf_user_embed, mf_item_embed, mlp_user_embed,
           mlp_item_embed, W0, b0, W1, b1, W2, b2, W3, b3, Wf, bf):
    xmf, xmlpu, xmlpi = _sc_gather(
        user.astype(jnp.int32), item.astype(jnp.int32),
        mf_user_embed, mf_item_embed, mlp_user_embed, mlp_item_embed)
    Ws = jnp.stack([W0, W1, W2, W3])
    bs = jnp.stack([b0, b1, b2, b3]).reshape(4, 1, MLP)
    wf = Wf.reshape(1, MF + MLP)
    out = pl.pallas_call(
        _tc_mlp_body,
        out_shape=jax.ShapeDtypeStruct((B, 1), jnp.float32),
    )(xmf, xmlpu, xmlpi, Ws, bs, wf, bf.reshape(1, 1))
    return out


# speedup vs baseline: 3.2080x; 3.2055x over previous
import functools

import jax
import jax.numpy as jnp
from jax import lax
from jax.experimental import pallas as pl
from jax.experimental.pallas import tpu as pltpu
from jax.experimental.pallas import tpu_sc as plsc

B = 16384
MF = 32
MLPH = 16
MLP = 2 * MLPH
NROWS = 1000000
NC, NS = 2, 16
NW = NC * NS
BPW = B // NW
NB = 8
NG = BPW // NB


def _sc_gather_body(user_hbm, item_hbm, mfu_hbm, mfi_hbm, mlu_hbm, mli_hbm,
                    xmf_hbm, xmlpu_hbm, xmlpi_hbm,
                    uidx, iidx, b_mfu, b_mfi, b_mlu, b_mli,
                    r_mf, r_mlu, r_mli, sem):
    wid = lax.axis_index("s") * NC + lax.axis_index("c")
    base = wid * BPW
    pltpu.sync_copy(user_hbm.at[pl.ds(base, BPW)], uidx)
    pltpu.sync_copy(item_hbm.at[pl.ds(base, BPW)], iidx)
    lanes = lax.iota(jnp.int32, 16)

    def group(g, carry):
        voff = pl.multiple_of((g // 2) * 16, 16)
        loff = (g % 2) * 8
        vu = uidx[pl.ds(voff, 16)]
        vi = iidx[pl.ds(voff, 16)]
        copies = []
        lanepos = []
        for l in range(NB):
            su = jnp.sum(jnp.where(lanes == loff + l, vu, 0))
            si = jnp.sum(jnp.where(lanes == loff + l, vi, 0))
            ou = pl.multiple_of((su // 128) * 128, 128)
            oi = pl.multiple_of((si // 128) * 128, 128)
            lanepos.append((su % 128, si % 128))
            du = pl.ds(ou, 128)
            di = pl.ds(oi, 128)
            copies.append(pltpu.async_copy(mfu_hbm.at[:, du], b_mfu.at[l], sem))
            copies.append(pltpu.async_copy(mfi_hbm.at[:, di], b_mfi.at[l], sem))
            copies.append(pltpu.async_copy(mlu_hbm.at[:, du], b_mlu.at[l], sem))
            copies.append(pltpu.async_copy(mli_hbm.at[:, di], b_mli.at[l], sem))
        for c in copies:
            c.wait()
        for l in range(NB):
            lu, li = lanepos[l]
            lvec = jnp.full((16,), l, jnp.int32)
            lu_v = jnp.full((16,), lu, jnp.int32)
            li_v = jnp.full((16,), li, jnp.int32)
            for d in (0, 16):
                dvec = d + lanes
                vmu = plsc.load_gather(b_mfu, [lvec, dvec, lu_v])
                vmi = plsc.load_gather(b_mfi, [lvec, dvec, li_v])
                r_mf[l, pl.ds(d, 16)] = vmu * vmi
            r_mlu[l, pl.ds(0, 16)] = plsc.load_gather(b_mlu, [lvec, lanes, lu_v])
            r_mli[l, pl.ds(0, 16)] = plsc.load_gather(b_mli, [lvec, lanes, li_v])
        out = pl.ds(pl.multiple_of(base + g * NB, NB), NB)
        pltpu.sync_copy(r_mf, xmf_hbm.at[out])
        pltpu.sync_copy(r_mlu, xmlpu_hbm.at[out])
        pltpu.sync_copy(r_mli, xmlpi_hbm.at[out])
        return carry

    lax.fori_loop(0, NG, group, 0)


_sc_gather = functools.partial(
    pl.kernel,
    out_type=(jax.ShapeDtypeStruct((B, MF), jnp.float32),
              jax.ShapeDtypeStruct((B, MLPH), jnp.float32),
              jax.ShapeDtypeStruct((B, MLPH), jnp.float32)),
    mesh=plsc.VectorSubcoreMesh(core_axis_name="c", subcore_axis_name="s"),
    scratch_types=[
        pltpu.VMEM((BPW,), jnp.int32),
        pltpu.VMEM((BPW,), jnp.int32),
        pltpu.VMEM((NB, MF, 128), jnp.float32),
        pltpu.VMEM((NB, MF, 128), jnp.float32),
        pltpu.VMEM((NB, MLPH, 128), jnp.float32),
        pltpu.VMEM((NB, MLPH, 128), jnp.float32),
        pltpu.VMEM((NB, MF), jnp.float32),
        pltpu.VMEM((NB, MLPH), jnp.float32),
        pltpu.VMEM((NB, MLPH), jnp.float32),
        pltpu.SemaphoreType.DMA,
    ],
    compiler_params=pltpu.CompilerParams(needs_layout_passes=False,
                                         use_tc_tiling_on_sc=True),
)(_sc_gather_body)


def _tc_mlp_body(xmf_ref, xmlpu_ref, xmlpi_ref, w_ref, b_ref, wf_ref, bf_ref,
                 out_ref):
    x = jnp.concatenate([xmlpu_ref[...], xmlpi_ref[...]], axis=1)
    for i in range(4):
        x = jnp.maximum(
            jnp.dot(x, w_ref[i], preferred_element_type=jnp.float32) + b_ref[i],
            0.0)
    wf = wf_ref[...]
    s = (jnp.sum(xmf_ref[...] * wf[:, :MF], axis=1, keepdims=True)
         + jnp.sum(x * wf[:, MF:], axis=1, keepdims=True)
         + bf_ref[0, 0])
    out_ref[...] = 1.0 / (1.0 + jnp.exp(-s))


def kernel(user, item, mf_user_embed, mf_item_embed, mlp_user_embed,
           mlp_item_embed, W0, b0, W1, b1, W2, b2, W3, b3, Wf, bf):
    xmf, xmlpu, xmlpi = _sc_gather(
        user.astype(jnp.int32), item.astype(jnp.int32),
        mf_user_embed.T, mf_item_embed.T, mlp_user_embed.T, mlp_item_embed.T)
    Ws = jnp.stack([W0, W1, W2, W3])
    bs = jnp.stack([b0, b1, b2, b3]).reshape(4, 1, MLP)
    wf = Wf.reshape(1, MF + MLP)
    out = pl.pallas_call(
        _tc_mlp_body,
        out_shape=jax.ShapeDtypeStruct((B, 1), jnp.float32),
    )(xmf, xmlpu, xmlpi, Ws, bs, wf, bf.reshape(1, 1))
    return out
